# baseline (speedup 1.0000x reference)
import functools

import jax
import jax.numpy as jnp
from jax import lax
from jax.experimental import pallas as pl
from jax.experimental.pallas import tpu as pltpu
from jax.experimental.pallas import tpu_sc as plsc

_NF_PAD = 1024


def _table_body(rot_ref, trans_ref, out_ref):
    nf = rot_ref.shape[1]
    pad = jnp.zeros((1, _NF_PAD - nf), jnp.float32)
    wx = jnp.concatenate([rot_ref[0:1, :], pad], axis=1)
    wy = jnp.concatenate([rot_ref[1:2, :], pad], axis=1)
    wz = jnp.concatenate([rot_ref[2:3, :], pad], axis=1)
    t0 = jnp.concatenate([trans_ref[0:1, :], pad], axis=1)
    t1 = jnp.concatenate([trans_ref[1:2, :], pad], axis=1)
    t2_ = jnp.concatenate([trans_ref[2:3, :], pad], axis=1)
    t2 = wx * wx + wy * wy + wz * wz
    theta = jnp.sqrt(jnp.maximum(t2, 1e-24))
    small = t2 < 1e-8
    a = jnp.where(small, 1.0 - t2 / 6.0, jnp.sin(theta) / theta)
    b = jnp.where(small, 0.5 - t2 / 24.0,
                  (1.0 - jnp.cos(theta)) / jnp.maximum(t2, 1e-24))
    axy = b * wx * wy
    axz = b * wx * wz
    ayz = b * wy * wz
    r00 = 1.0 - b * (wy * wy + wz * wz)
    r11 = 1.0 - b * (wx * wx + wz * wz)
    r22 = 1.0 - b * (wx * wx + wy * wy)
    r01 = axy - a * wz
    r10 = axy + a * wz
    r02 = axz + a * wy
    r20 = axz - a * wy
    r12 = ayz - a * wx
    r21 = ayz + a * wx
    out_ref[...] = jnp.concatenate(
        [r00, r01, r02, r10, r11, r12, r20, r21, r22, t0, t1, t2_], axis=0)


def _make_table(rot_t, trans_t):
    return pl.pallas_call(
        _table_body,
        out_shape=jax.ShapeDtypeStruct((12, _NF_PAD), jnp.float32),
    )(rot_t, trans_t)


def _sc_apply(table_flat, idx, mask_t, rays_t, n_rays):
    info = plsc.get_sparse_core_info()
    nc, ns = info.num_cores, info.num_subcores
    nw = nc * ns
    ch = n_rays // nw
    half = ch // 2
    id_col = _NF_PAD - 8
    mesh = plsc.VectorSubcoreMesh(core_axis_name="c", subcore_axis_name="s")

    @functools.partial(
        pl.kernel,
        out_type=jax.ShapeDtypeStruct((8, n_rays), jnp.float32),
        mesh=mesh,
        scratch_types=[
            pltpu.VMEM_SHARED((12 * _NF_PAD,), jnp.float32),
            pltpu.VMEM((12 * _NF_PAD,), jnp.float32),
            pltpu.VMEM((ch,), jnp.int32),
            pltpu.VMEM((1, ch), jnp.int32),
            pltpu.VMEM((8, ch), jnp.float32),
            pltpu.VMEM((8, ch), jnp.float32),
            pltpu.SemaphoreType.DMA,
            pltpu.SemaphoreType.DMA,
            pltpu.SemaphoreType.DMA,
            pltpu.SemaphoreType.DMA,
            pltpu.SemaphoreType.DMA,
            pltpu.SemaphoreType.DMA,
        ],
        compiler_params=pltpu.CompilerParams(needs_layout_passes=False),
    )
    def body(tab_hbm, idx_hbm, mask_hbm, rays_hbm, out_hbm,
             tab_sh, tab_v, idx_v, mask_v, rays_v, out_v,
             sem_tab, sem_q0, sem_q1, sem_q2, sem_q3, sem_out):
        wid = lax.axis_index("s") * nc + lax.axis_index("c")
        base = wid * ch
        qs = ch // 4
        sem_q = [sem_q0, sem_q1, sem_q2, sem_q3]

        @pl.when(lax.axis_index("s") == 0)
        def _stage_table():
            pltpu.sync_copy(tab_hbm, tab_sh)

        h_in = []
        for q in range(4):
            o = q * qs
            h_in.append([
                pltpu.async_copy(idx_hbm.at[pl.ds(base + o, qs)],
                                 idx_v.at[pl.ds(o, qs)], sem_q[q]),
                pltpu.async_copy(mask_hbm.at[:, pl.ds(base + o, qs)],
                                 mask_v.at[:, pl.ds(o, qs)], sem_q[q]),
                pltpu.async_copy(rays_hbm.at[:, pl.ds(base + o, qs)],
                                 rays_v.at[:, pl.ds(o, qs)], sem_q[q]),
            ])
        plsc.subcore_barrier()
        pltpu.async_copy(tab_sh, tab_v, sem_tab).wait()

        def step(s):
            fidx = idx_v[pl.ds(s, 16)]
            m = mask_v[0, pl.ds(s, 16)]
            fidx = jnp.where(m == 1, fidx, id_col)
            ray = [rays_v[c, pl.ds(s, 16)] for c in range(8)]
            g = [plsc.load_gather(tab_v, [fidx + (c * _NF_PAD)])
                 for c in range(12)]
            d0, d1, d2 = ray[3], ray[4], ray[5]
            outs = [
                ray[0] + g[9],
                ray[1] + g[10],
                ray[2] + g[11],
                g[0] * d0 + g[1] * d1 + g[2] * d2,
                g[3] * d0 + g[4] * d1 + g[5] * d2,
                g[6] * d0 + g[7] * d1 + g[8] * d2,
                ray[6],
                ray[7],
            ]
            for c in range(8):
                out_v[c, pl.ds(s, 16)] = outs[c]

        h_out = []
        for q in range(4):
            o = q * qs
            for h in h_in[q]:
                h.wait()
            plsc.parallel_loop(o, o + qs, 16, unroll=8)(step)
            if q == 0:
                @pl.when(wid == 0)
                def _fix_head():
                    iota = lax.iota(jnp.int32, 16)
                    keep = iota >= 6
                    fz = jnp.zeros((16,), jnp.float32)
                    for c in (6, 7):
                        out_v[c, pl.ds(0, 16)] = jnp.where(
                            keep, out_v[c, pl.ds(0, 16)], fz)
            if q < 3:
                h_out.append(pltpu.async_copy(
                    out_v.at[:, pl.ds(o, qs)],
                    out_hbm.at[:, pl.ds(base + o, qs)], sem_out))
        pltpu.sync_copy(out_v.at[:, pl.ds(3 * qs, qs)],
                        out_hbm.at[:, pl.ds(base + 3 * qs, qs)])
        for h in h_out:
            h.wait()

    return body(table_flat, idx, mask_t, rays_t)


def kernel(image_indices, rays, depth_mask, rot_dict, trans_dict):
    n = rays.shape[0]
    idx = image_indices.astype(jnp.int32)
    table = _make_table(rot_dict.astype(jnp.float32).T,
                        trans_dict.astype(jnp.float32).T)
    out_t = _sc_apply(table.reshape(12 * _NF_PAD), idx,
                      depth_mask.astype(jnp.int32).T,
                      rays.astype(jnp.float32).T, n)
    return out_t.T


# speedup vs baseline: 1.2196x; 1.2196x over previous
import functools

import jax
import jax.numpy as jnp
from jax import lax
from jax.experimental import pallas as pl
from jax.experimental.pallas import tpu as pltpu
from jax.experimental.pallas import tpu_sc as plsc

_NF_PAD = 1024


def _table_body(rot_ref, trans_ref, out_ref):
    nf = rot_ref.shape[1]
    pad = jnp.zeros((1, _NF_PAD - nf), jnp.float32)
    wx = jnp.concatenate([rot_ref[0:1, :], pad], axis=1)
    wy = jnp.concatenate([rot_ref[1:2, :], pad], axis=1)
    wz = jnp.concatenate([rot_ref[2:3, :], pad], axis=1)
    t0 = jnp.concatenate([trans_ref[0:1, :], pad], axis=1)
    t1 = jnp.concatenate([trans_ref[1:2, :], pad], axis=1)
    t2_ = jnp.concatenate([trans_ref[2:3, :], pad], axis=1)
    t2 = wx * wx + wy * wy + wz * wz
    theta = jnp.sqrt(jnp.maximum(t2, 1e-24))
    small = t2 < 1e-8
    a = jnp.where(small, 1.0 - t2 / 6.0, jnp.sin(theta) / theta)
    b = jnp.where(small, 0.5 - t2 / 24.0,
                  (1.0 - jnp.cos(theta)) / jnp.maximum(t2, 1e-24))
    axy = b * wx * wy
    axz = b * wx * wz
    ayz = b * wy * wz
    r00 = 1.0 - b * (wy * wy + wz * wz)
    r11 = 1.0 - b * (wx * wx + wz * wz)
    r22 = 1.0 - b * (wx * wx + wy * wy)
    r01 = axy - a * wz
    r10 = axy + a * wz
    r02 = axz + a * wy
    r20 = axz - a * wy
    r12 = ayz - a * wx
    r21 = ayz + a * wx
    out_ref[...] = jnp.concatenate(
        [r00, r01, r02, r10, r11, r12, r20, r21, r22, t0, t1, t2_], axis=0)


def _make_table(rot_t, trans_t):
    return pl.pallas_call(
        _table_body,
        out_shape=jax.ShapeDtypeStruct((12, _NF_PAD), jnp.float32),
    )(rot_t, trans_t)


def _sc_apply(table_flat, idx, mask_t, rays_t, n_rays):
    info = plsc.get_sparse_core_info()
    nc, ns = info.num_cores, info.num_subcores
    nw = nc * ns
    ch = n_rays // nw
    half = ch // 2
    id_col = _NF_PAD - 8
    mesh = plsc.VectorSubcoreMesh(core_axis_name="c", subcore_axis_name="s")

    @functools.partial(
        pl.kernel,
        out_type=jax.ShapeDtypeStruct((8, n_rays), jnp.float32),
        mesh=mesh,
        scratch_types=[
            pltpu.VMEM_SHARED((12 * _NF_PAD,), jnp.float32),
            pltpu.VMEM((12 * _NF_PAD,), jnp.float32),
            pltpu.VMEM((ch,), jnp.int32),
            pltpu.VMEM((1, ch), jnp.int32),
            pltpu.VMEM((8, ch), jnp.float32),
            pltpu.VMEM((8, ch), jnp.float32),
            pltpu.SemaphoreType.DMA,
            pltpu.SemaphoreType.DMA,
            pltpu.SemaphoreType.DMA,
            pltpu.SemaphoreType.DMA,
            pltpu.SemaphoreType.DMA,
            pltpu.SemaphoreType.DMA,
        ],
        compiler_params=pltpu.CompilerParams(needs_layout_passes=False),
    )
    def body(tab_hbm, idx_hbm, mask_hbm, rays_hbm, out_hbm,
             tab_sh, tab_v, idx_v, mask_v, rays_v, out_v,
             sem_tab, sem_q0, sem_q1, sem_q2, sem_q3, sem_out):
        wid = lax.axis_index("s") * nc + lax.axis_index("c")
        base = wid * ch
        qs = ch // 4
        sem_q = [sem_q0, sem_q1, sem_q2, sem_q3]

        @pl.when(lax.axis_index("s") == 0)
        def _stage_table():
            pltpu.sync_copy(tab_hbm, tab_sh)

        h_in = []
        for q in range(4):
            o = q * qs
            h_in.append([
                pltpu.async_copy(idx_hbm.at[pl.ds(base + o, qs)],
                                 idx_v.at[pl.ds(o, qs)], sem_q[q]),
                pltpu.async_copy(mask_hbm.at[:, pl.ds(base + o, qs)],
                                 mask_v.at[:, pl.ds(o, qs)], sem_q[q]),
                pltpu.async_copy(rays_hbm.at[:, pl.ds(base + o, qs)],
                                 rays_v.at[:, pl.ds(o, qs)], sem_q[q]),
            ])
        plsc.subcore_barrier()
        pltpu.async_copy(tab_sh, tab_v, sem_tab).wait()

        def step(s):
            fidx = idx_v[pl.ds(s, 16)]
            m = mask_v[0, pl.ds(s, 16)]
            fidx = jnp.where(m == 1, fidx, id_col)
            ray = [rays_v[c, pl.ds(s, 16)] for c in range(8)]
            g = [plsc.load_gather(tab_v, [fidx + (c * _NF_PAD)])
                 for c in range(12)]
            d0, d1, d2 = ray[3], ray[4], ray[5]
            outs = [
                ray[0] + g[9],
                ray[1] + g[10],
                ray[2] + g[11],
                g[0] * d0 + g[1] * d1 + g[2] * d2,
                g[3] * d0 + g[4] * d1 + g[5] * d2,
                g[6] * d0 + g[7] * d1 + g[8] * d2,
                ray[6],
                ray[7],
            ]
            for c in range(8):
                out_v[c, pl.ds(s, 16)] = outs[c]

        h_out = []
        for q in range(4):
            o = q * qs
            for h in h_in[q]:
                h.wait()
            plsc.parallel_loop(o, o + qs, 16, unroll=2)(step)
            if q == 0:
                @pl.when(wid == 0)
                def _fix_head():
                    iota = lax.iota(jnp.int32, 16)
                    keep = iota >= 6
                    fz = jnp.zeros((16,), jnp.float32)
                    for c in (6, 7):
                        out_v[c, pl.ds(0, 16)] = jnp.where(
                            keep, out_v[c, pl.ds(0, 16)], fz)
            if q < 3:
                h_out.append(pltpu.async_copy(
                    out_v.at[:, pl.ds(o, qs)],
                    out_hbm.at[:, pl.ds(base + o, qs)], sem_out))
        pltpu.sync_copy(out_v.at[:, pl.ds(3 * qs, qs)],
                        out_hbm.at[:, pl.ds(base + 3 * qs, qs)])
        for h in h_out:
            h.wait()

    return body(table_flat, idx, mask_t, rays_t)


def kernel(image_indices, rays, depth_mask, rot_dict, trans_dict):
    n = rays.shape[0]
    idx = image_indices.astype(jnp.int32)
    table = _make_table(rot_dict.astype(jnp.float32).T,
                        trans_dict.astype(jnp.float32).T)
    out_t = _sc_apply(table.reshape(12 * _NF_PAD), idx,
                      depth_mask.astype(jnp.int32).T,
                      rays.astype(jnp.float32).T, n)
    return out_t.T


# speedup vs baseline: 1.2217x; 1.0018x over previous
import functools

import jax
import jax.numpy as jnp
from jax import lax
from jax.experimental import pallas as pl
from jax.experimental.pallas import tpu as pltpu
from jax.experimental.pallas import tpu_sc as plsc

_NF_PAD = 1024


def _table_body(rot_ref, trans_ref, out_ref):
    nf = rot_ref.shape[1]
    pad = jnp.zeros((1, _NF_PAD - nf), jnp.float32)
    wx = jnp.concatenate([rot_ref[0:1, :], pad], axis=1)
    wy = jnp.concatenate([rot_ref[1:2, :], pad], axis=1)
    wz = jnp.concatenate([rot_ref[2:3, :], pad], axis=1)
    t0 = jnp.concatenate([trans_ref[0:1, :], pad], axis=1)
    t1 = jnp.concatenate([trans_ref[1:2, :], pad], axis=1)
    t2_ = jnp.concatenate([trans_ref[2:3, :], pad], axis=1)
    t2 = wx * wx + wy * wy + wz * wz
    theta = jnp.sqrt(jnp.maximum(t2, 1e-24))
    small = t2 < 1e-8
    a = jnp.where(small, 1.0 - t2 / 6.0, jnp.sin(theta) / theta)
    b = jnp.where(small, 0.5 - t2 / 24.0,
                  (1.0 - jnp.cos(theta)) / jnp.maximum(t2, 1e-24))
    axy = b * wx * wy
    axz = b * wx * wz
    ayz = b * wy * wz
    r00 = 1.0 - b * (wy * wy + wz * wz)
    r11 = 1.0 - b * (wx * wx + wz * wz)
    r22 = 1.0 - b * (wx * wx + wy * wy)
    r01 = axy - a * wz
    r10 = axy + a * wz
    r02 = axz + a * wy
    r20 = axz - a * wy
    r12 = ayz - a * wx
    r21 = ayz + a * wx
    out_ref[...] = jnp.concatenate(
        [r00, r01, r02, r10, r11, r12, r20, r21, r22, t0, t1, t2_], axis=0)


def _make_table(rot_t, trans_t):
    return pl.pallas_call(
        _table_body,
        out_shape=jax.ShapeDtypeStruct((12, _NF_PAD), jnp.float32),
    )(rot_t, trans_t)


def _sc_apply(table_flat, idx, mask_t, rays_t, n_rays):
    info = plsc.get_sparse_core_info()
    nc, ns = info.num_cores, info.num_subcores
    nw = nc * ns
    ch = n_rays // nw
    half = ch // 2
    id_col = _NF_PAD - 8
    mesh = plsc.VectorSubcoreMesh(core_axis_name="c", subcore_axis_name="s")

    @functools.partial(
        pl.kernel,
        out_type=jax.ShapeDtypeStruct((8, n_rays), jnp.float32),
        mesh=mesh,
        scratch_types=[
            pltpu.VMEM_SHARED((12 * _NF_PAD,), jnp.float32),
            pltpu.VMEM((12 * _NF_PAD,), jnp.float32),
            pltpu.VMEM((ch,), jnp.int32),
            pltpu.VMEM((1, ch), jnp.int32),
            pltpu.VMEM((8, ch), jnp.float32),
            pltpu.VMEM((8, ch), jnp.float32),
            pltpu.SemaphoreType.DMA,
            pltpu.SemaphoreType.DMA,
            pltpu.SemaphoreType.DMA,
            pltpu.SemaphoreType.DMA,
            pltpu.SemaphoreType.DMA,
            pltpu.SemaphoreType.DMA,
        ],
        compiler_params=pltpu.CompilerParams(needs_layout_passes=False),
    )
    def body(tab_hbm, idx_hbm, mask_hbm, rays_hbm, out_hbm,
             tab_sh, tab_v, idx_v, mask_v, rays_v, out_v,
             sem_tab, sem_q0, sem_q1, sem_q2, sem_q3, sem_out):
        wid = lax.axis_index("s") * nc + lax.axis_index("c")
        base = wid * ch
        qs = ch // 4
        sem_q = [sem_q0, sem_q1, sem_q2, sem_q3]

        @pl.when(lax.axis_index("s") == 0)
        def _stage_table():
            pltpu.sync_copy(tab_hbm, tab_sh)

        h_in = []
        for q in range(4):
            o = q * qs
            h_in.append([
                pltpu.async_copy(idx_hbm.at[pl.ds(base + o, qs)],
                                 idx_v.at[pl.ds(o, qs)], sem_q[q]),
                pltpu.async_copy(mask_hbm.at[:, pl.ds(base + o, qs)],
                                 mask_v.at[:, pl.ds(o, qs)], sem_q[q]),
                pltpu.async_copy(rays_hbm.at[:, pl.ds(base + o, qs)],
                                 rays_v.at[:, pl.ds(o, qs)], sem_q[q]),
            ])
        plsc.subcore_barrier()
        pltpu.async_copy(tab_sh, tab_v, sem_tab).wait()

        def step(s):
            fidx = idx_v[pl.ds(s, 16)]
            m = mask_v[0, pl.ds(s, 16)]
            fidx = jnp.where(m == 1, fidx, id_col)
            ray = [rays_v[c, pl.ds(s, 16)] for c in range(8)]
            g = [plsc.load_gather(tab_v, [fidx + (c * _NF_PAD)])
                 for c in range(12)]
            d0, d1, d2 = ray[3], ray[4], ray[5]
            outs = [
                ray[0] + g[9],
                ray[1] + g[10],
                ray[2] + g[11],
                g[0] * d0 + g[1] * d1 + g[2] * d2,
                g[3] * d0 + g[4] * d1 + g[5] * d2,
                g[6] * d0 + g[7] * d1 + g[8] * d2,
                ray[6],
                ray[7],
            ]
            for c in range(8):
                out_v[c, pl.ds(s, 16)] = outs[c]

        h_out = []
        for q in range(4):
            o = q * qs
            for h in h_in[q]:
                h.wait()
            plsc.parallel_loop(o, o + qs, 16, unroll=1)(step)
            if q == 0:
                @pl.when(wid == 0)
                def _fix_head():
                    iota = lax.iota(jnp.int32, 16)
                    keep = iota >= 6
                    fz = jnp.zeros((16,), jnp.float32)
                    for c in (6, 7):
                        out_v[c, pl.ds(0, 16)] = jnp.where(
                            keep, out_v[c, pl.ds(0, 16)], fz)
            if q < 3:
                h_out.append(pltpu.async_copy(
                    out_v.at[:, pl.ds(o, qs)],
                    out_hbm.at[:, pl.ds(base + o, qs)], sem_out))
        pltpu.sync_copy(out_v.at[:, pl.ds(3 * qs, qs)],
                        out_hbm.at[:, pl.ds(base + 3 * qs, qs)])
        for h in h_out:
            h.wait()

    return body(table_flat, idx, mask_t, rays_t)


def kernel(image_indices, rays, depth_mask, rot_dict, trans_dict):
    n = rays.shape[0]
    idx = image_indices.astype(jnp.int32)
    table = _make_table(rot_dict.astype(jnp.float32).T,
                        trans_dict.astype(jnp.float32).T)
    out_t = _sc_apply(table.reshape(12 * _NF_PAD), idx,
                      depth_mask.astype(jnp.int32).T,
                      rays.astype(jnp.float32).T, n)
    return out_t.T
